# baseline (speedup 1.0000x reference)
import functools

import numpy as np

import jax
import jax.numpy as jnp
from jax import lax
from jax.experimental import pallas as pl
from jax.experimental.pallas import tpu as pltpu
from jax.experimental.pallas import tpu_sc as plsc

_N = 10000
_D = 128
_NNZ = 160000
_NLIVE = 3
_NT = 32
_CH = 128
_EPT = 5120
_EPTOT = _EPT * _NT
_NCH = _EPT // _CH
_NP = 10240
_RPS = _NP // 16
_DCH = 128

def _mm_body(x_ref, w_ref, o_ref):
    o_ref[:, :] = jnp.dot(x_ref[:, :], w_ref[:, :],
                          preferred_element_type=jnp.float32)


def _matmul(x, w):
    return pl.pallas_call(
        _mm_body,
        grid=(10,),
        in_specs=[pl.BlockSpec((_N // 10, _D), lambda i: (i, 0)),
                  pl.BlockSpec((_D, _D), lambda i: (0, 0))],
        out_specs=pl.BlockSpec((_N // 10, _D), lambda i: (i, 0)),
        out_shape=jax.ShapeDtypeStruct((_N, _D), jnp.float32),
    )(x, w)


def _merge_body(a_ref, o_ref):
    o_ref[:, :] = a_ref[0] + a_ref[1]


def _merge(yp):
    rows = _NLIVE * _NP
    blk = 1024
    return pl.pallas_call(
        _merge_body,
        grid=(rows // blk,),
        in_specs=[pl.BlockSpec((2, blk, _D), lambda i: (0, i, 0))],
        out_specs=pl.BlockSpec((blk, _D), lambda i: (i, 0)),
        out_shape=jax.ShapeDtypeStruct((rows, _D), jnp.float32),
    )(yp)


def _final_body(a_ref, b_ref, o_ref):
    o_ref[:, :] = a_ref[0] + a_ref[1] + b_ref[:, :]


def _final(op, bias2d):
    blk = 1000
    return pl.pallas_call(
        _final_body,
        grid=(_N // blk,),
        in_specs=[pl.BlockSpec((2, blk, _D), lambda i: (0, i, 0)),
                  pl.BlockSpec((1, _D), lambda i: (0, 0))],
        out_specs=pl.BlockSpec((blk, _D), lambda i: (i, 0)),
        out_shape=jax.ShapeDtypeStruct((_N, _D), jnp.float32),
    )(op, bias2d)


_MESH = dict(core_axis_name="c", subcore_axis_name="s")
_SC_PARAMS = dict(
    compiler_params=pltpu.CompilerParams(needs_layout_passes=False))


def _zero_buf(buf):
    def zrow(i, carry):
        for j in range(_D // 16):
            buf[i, pl.ds(j * 16, 16)] = jnp.zeros((16,), jnp.float32)
        return carry
    lax.fori_loop(0, _DCH, zrow, 0)


def _scale_rows(gbuf, sbuf, vals_v):
    def erow(e, carry):
        e16 = jnp.full((16,), 0, jnp.int32) + e
        s16 = plsc.load_gather(vals_v, [e16])
        for g in range(_D // 16):
            sbuf[e, pl.ds(g * 16, 16)] = gbuf[e, pl.ds(g * 16, 16)] * s16
        return carry
    lax.fori_loop(0, _CH, erow, 0)


_IDXBUFS = lambda: [pltpu.VMEM((_CH,), jnp.int32),
                    pltpu.VMEM((_CH,), jnp.int32),
                    pltpu.VMEM((_CH,), jnp.float32)]


def _sc_stage1(hp, rows_p, cols_p, vals_p, filt_flat):
    mesh = plsc.VectorSubcoreMesh(**_MESH)

    @functools.partial(
        pl.kernel,
        out_type=jax.ShapeDtypeStruct((2, _NLIVE, _NP, _D), jnp.float32),
        mesh=mesh,
        scratch_types=[
            pltpu.VMEM_SHARED((_NP, _D), jnp.float32),
            *_IDXBUFS(),
            pltpu.VMEM((_CH, _D), jnp.float32),
            pltpu.VMEM((_CH, _D), jnp.float32),
            pltpu.VMEM((_N,), jnp.float32),
            pltpu.SemaphoreType.DMA,
        ],
        **_SC_PARAMS,
    )
    def k(hp_hbm, erows_hbm, ecols_hbm, evals_hbm, filt_hbm, yp_hbm,
          ysp, rows_v, cols_v, vals_v, gbuf, sbuf, filt_v, sem):
        cid = lax.axis_index("c")
        sid = lax.axis_index("s")
        tid = cid * 16 + sid
        base = tid * _EPT

        _zero_buf(sbuf)
        for i in range(_RPS // _DCH):
            pltpu.sync_copy(sbuf, ysp.at[pl.ds(sid * _RPS + i * _DCH, _DCH)])
        plsc.subcore_barrier()

        for mm in range(_NLIVE):
            pltpu.sync_copy(filt_hbm.at[pl.ds((mm + 1) * _N, _N)], filt_v)

            def chunk(g, carry):
                off = mm * _EPTOT + base + g * _CH
                pltpu.sync_copy(erows_hbm.at[pl.ds(off, _CH)], rows_v)
                pltpu.sync_copy(ecols_hbm.at[pl.ds(off, _CH)], cols_v)
                pltpu.sync_copy(evals_hbm.at[pl.ds(off, _CH)], vals_v)
                pltpu.async_copy(hp_hbm.at[cols_v], gbuf, sem).wait()
                for j in range(_CH // 16):
                    r16 = rows_v[pl.ds(j * 16, 16)]
                    f16 = plsc.load_gather(filt_v, [r16])
                    vals_v[pl.ds(j * 16, 16)] = vals_v[pl.ds(j * 16, 16)] * f16
                _scale_rows(gbuf, sbuf, vals_v)
                pltpu.sync_copy(sbuf, ysp.at[rows_v], add=True)
                return carry

            lax.fori_loop(0, _NCH, chunk, 0)

            plsc.subcore_barrier()
            _zero_buf(sbuf)
            for i in range(_RPS // _DCH):
                start = sid * _RPS + i * _DCH
                pltpu.sync_copy(ysp.at[pl.ds(start, _DCH)],
                                yp_hbm.at[cid, mm, pl.ds(start, _DCH)])
                pltpu.sync_copy(sbuf, ysp.at[pl.ds(start, _DCH)])
            plsc.subcore_barrier()

    return k(hp, rows_p, cols_p, vals_p, filt_flat)


def _sc_stage2(ymp, rows_p, cols_p, vals_p):
    mesh = plsc.VectorSubcoreMesh(**_MESH)

    @functools.partial(
        pl.kernel,
        out_type=jax.ShapeDtypeStruct((2, _NP, _D), jnp.float32),
        mesh=mesh,
        scratch_types=[
            pltpu.VMEM_SHARED((_NP, _D), jnp.float32),
            *_IDXBUFS(),
            pltpu.VMEM((_CH, _D), jnp.float32),
            pltpu.VMEM((_CH, _D), jnp.float32),
            pltpu.SemaphoreType.DMA,
        ],
        **_SC_PARAMS,
    )
    def k(ymp_hbm, erows_hbm, ecols_hbm, evals_hbm, op_hbm,
          osp, rows_v, cols_v, vals_v, gbuf, sbuf, sem):
        cid = lax.axis_index("c")
        sid = lax.axis_index("s")
        tid = cid * 16 + sid
        base = tid * _EPT

        _zero_buf(sbuf)
        for i in range(_RPS // _DCH):
            pltpu.sync_copy(sbuf, osp.at[pl.ds(sid * _RPS + i * _DCH, _DCH)])
        plsc.subcore_barrier()

        for mm in range(_NLIVE):
            yoff = mm * _NP

            def chunk(g, carry):
                off = mm * _EPTOT + base + g * _CH
                pltpu.sync_copy(erows_hbm.at[pl.ds(off, _CH)], rows_v)
                pltpu.sync_copy(ecols_hbm.at[pl.ds(off, _CH)], cols_v)
                pltpu.sync_copy(evals_hbm.at[pl.ds(off, _CH)], vals_v)
                for j in range(_CH // 16):
                    c16 = cols_v[pl.ds(j * 16, 16)]
                    cols_v[pl.ds(j * 16, 16)] = c16 + yoff
                pltpu.async_copy(ymp_hbm.at[cols_v], gbuf, sem).wait()
                _scale_rows(gbuf, sbuf, vals_v)
                pltpu.sync_copy(sbuf, osp.at[rows_v], add=True)
                return carry

            lax.fori_loop(0, _NCH, chunk, 0)

        plsc.subcore_barrier()
        for i in range(_RPS // _DCH):
            start = sid * _RPS + i * _DCH
            pltpu.sync_copy(osp.at[pl.ds(start, _DCH)],
                            op_hbm.at[cid, pl.ds(start, _DCH)])

    return k(ymp, rows_p, cols_p, vals_p)


def kernel(x, d_indices, d_values, weight, filt, bias):
    pad = ((0, 0), (0, _EPTOT - _NNZ))
    rows_p = jnp.pad(d_indices[1:, 0, :], pad).reshape(-1)
    cols_p = jnp.pad(d_indices[1:, 1, :], pad).reshape(-1)
    vals_p = jnp.pad(d_values[1:], pad).reshape(-1)
    h = _matmul(x, weight)
    yp = _sc_stage1(h, rows_p, cols_p, vals_p, filt.reshape(-1))
    ym = _merge(yp.reshape(2, _NLIVE * _NP, _D))
    op = _sc_stage2(ym, rows_p, cols_p, vals_p)
    return _final(op, bias.reshape(1, _D))


# speedup vs baseline: 2.0309x; 2.0309x over previous
import functools

import numpy as np

import jax
import jax.numpy as jnp
from jax import lax
from jax.experimental import pallas as pl
from jax.experimental.pallas import tpu as pltpu
from jax.experimental.pallas import tpu_sc as plsc

_N = 10000
_D = 128
_NNZ = 160000
_NLIVE = 3
_NT = 32
_CH = 128
_EPT = 5120
_EPTOT = _EPT * _NT
_NCH = _EPT // _CH
_NP = 10240
_RPS = _NP // 16
_DCH = 128
_IBR = 8


def _mm_body(x_ref, w_ref, o_ref):
    o_ref[:, :] = jnp.dot(x_ref[:, :], w_ref[:, :],
                          preferred_element_type=jnp.float32)


def _matmul(x, w):
    return pl.pallas_call(
        _mm_body,
        grid=(10,),
        in_specs=[pl.BlockSpec((_N // 10, _D), lambda i: (i, 0)),
                  pl.BlockSpec((_D, _D), lambda i: (0, 0))],
        out_specs=pl.BlockSpec((_N // 10, _D), lambda i: (i, 0)),
        out_shape=jax.ShapeDtypeStruct((_N, _D), jnp.float32),
    )(x, w)


def _merge_body(a_ref, f_ref, o_ref):
    o_ref[:, :] = (a_ref[0] + a_ref[1]) * f_ref[:, :]


def _merge_scale(yp, filtp):
    rows = _NLIVE * _NP
    blk = 1024
    return pl.pallas_call(
        _merge_body,
        grid=(rows // blk,),
        in_specs=[pl.BlockSpec((2, blk, _D), lambda i: (0, i, 0)),
                  pl.BlockSpec((blk, 1), lambda i: (i, 0))],
        out_specs=pl.BlockSpec((blk, _D), lambda i: (i, 0)),
        out_shape=jax.ShapeDtypeStruct((rows, _D), jnp.float32),
    )(yp, filtp)


def _final_body(a_ref, b_ref, o_ref):
    o_ref[:, :] = a_ref[0] + a_ref[1] + b_ref[:, :]


def _final(op, bias2d):
    blk = 1000
    return pl.pallas_call(
        _final_body,
        grid=(_N // blk,),
        in_specs=[pl.BlockSpec((2, blk, _D), lambda i: (0, i, 0)),
                  pl.BlockSpec((1, _D), lambda i: (0, 0))],
        out_specs=pl.BlockSpec((blk, _D), lambda i: (i, 0)),
        out_shape=jax.ShapeDtypeStruct((_N, _D), jnp.float32),
    )(op, bias2d)


_MESH = dict(core_axis_name="c", subcore_axis_name="s")
_SC_PARAMS = dict(
    compiler_params=pltpu.CompilerParams(needs_layout_passes=False))


def _zero_buf(buf):
    def zrow(i, carry):
        for j in range(_D // 16):
            buf[i, pl.ds(j * 16, 16)] = jnp.zeros((16,), jnp.float32)
        return carry
    lax.fori_loop(0, _DCH, zrow, 0)


def _scale_rows(gbuf, ibuf):
    def erow(e, carry):
        e16 = jnp.full((16,), 0, jnp.int32) + e
        vi = plsc.load_gather(ibuf.at[2], [e16])
        s16 = plsc.bitcast(vi, jnp.float32)
        for g in range(_D // 16):
            gbuf[e, pl.ds(g * 16, 16)] = gbuf[e, pl.ds(g * 16, 16)] * s16
        return carry
    lax.fori_loop(0, _CH, erow, 0)


def _sc_bufs():
    return [
        pltpu.VMEM((_IBR, _CH), jnp.int32),
        pltpu.VMEM((_IBR, _CH), jnp.int32),
        pltpu.VMEM((_CH, _D), jnp.float32),
        pltpu.VMEM((_CH, _D), jnp.float32),
        pltpu.SemaphoreType.DMA,
        pltpu.SemaphoreType.DMA,
    ]


def _edge_pipeline(src_hbm, idx_hbm, acc, blk0, ibufs, gbufs, sems):
    i0, i1 = ibufs
    g0, g1 = gbufs
    s0, s1 = sems

    def load_idx(c, ibuf):
        pltpu.sync_copy(idx_hbm.at[pl.ds((blk0 + c) * _IBR, _IBR)], ibuf)

    def issue(ibuf, gbuf, sem):
        pltpu.async_copy(src_hbm.at[ibuf.at[1]], gbuf, sem)

    def drain(gbuf, sem):
        pltpu.make_async_copy(src_hbm.at[pl.ds(0, _CH)], gbuf, sem).wait()

    def process(ibuf, gbuf):
        _scale_rows(gbuf, ibuf)
        pltpu.sync_copy(gbuf, acc.at[ibuf.at[0]], add=True)

    load_idx(0, i0)
    issue(i0, g0, s0)
    load_idx(1, i1)
    issue(i1, g1, s1)

    def pair(k, carry):
        c = 2 * k
        drain(g0, s0)
        process(i0, g0)
        load_idx(c + 2, i0)
        issue(i0, g0, s0)
        drain(g1, s1)
        process(i1, g1)
        load_idx(c + 3, i1)
        issue(i1, g1, s1)
        return carry

    lax.fori_loop(0, _NCH // 2 - 1, pair, 0)

    drain(g0, s0)
    process(i0, g0)
    drain(g1, s1)
    process(i1, g1)


def _sc_stage(src, idx, out_shape, dump):
    mesh = plsc.VectorSubcoreMesh(**_MESH)

    @functools.partial(
        pl.kernel,
        out_type=jax.ShapeDtypeStruct(out_shape, jnp.float32),
        mesh=mesh,
        scratch_types=[
            pltpu.VMEM_SHARED((_NP, _D), jnp.float32),
            *_sc_bufs(),
        ],
        **_SC_PARAMS,
    )
    def k(src_hbm, idx_hbm, out_hbm, acc, i0, i1, g0, g1, s0, s1):
        cid = lax.axis_index("c")
        sid = lax.axis_index("s")
        tid = cid * 16 + sid

        _zero_buf(g0)
        for i in range(_RPS // _DCH):
            pltpu.sync_copy(g0, acc.at[pl.ds(sid * _RPS + i * _DCH, _DCH)])
        plsc.subcore_barrier()

        for mm in range(_NLIVE):
            blk0 = (mm * _NT + tid) * _NCH
            _edge_pipeline(src_hbm, idx_hbm, acc, blk0,
                           (i0, i1), (g0, g1), (s0, s1))
            plsc.subcore_barrier()
            dump(mm, cid, sid, acc, g0, out_hbm)
            plsc.subcore_barrier()

    return k(src, idx)


def _dump_per_op(mm, cid, sid, acc, zbuf, out_hbm):
    _zero_buf(zbuf)
    for i in range(_RPS // _DCH):
        start = sid * _RPS + i * _DCH
        pltpu.sync_copy(acc.at[pl.ds(start, _DCH)],
                        out_hbm.at[cid, mm, pl.ds(start, _DCH)])
        pltpu.sync_copy(zbuf, acc.at[pl.ds(start, _DCH)])


def _dump_final(mm, cid, sid, acc, zbuf, out_hbm):
    if mm == _NLIVE - 1:
        for i in range(_RPS // _DCH):
            start = sid * _RPS + i * _DCH
            pltpu.sync_copy(acc.at[pl.ds(start, _DCH)],
                            out_hbm.at[cid, pl.ds(start, _DCH)])


def _pack_idx(rows, cols, vals):
    vbits = lax.bitcast_convert_type(vals, jnp.int32)
    trio = jnp.stack([rows, cols, vbits], axis=1)
    trio = trio.reshape(_NLIVE, 3, _NT * _NCH, _CH)
    trio = trio.transpose(0, 2, 1, 3)
    pad = jnp.zeros((_NLIVE, _NT * _NCH, _IBR - 3, _CH), jnp.int32)
    blocks = jnp.concatenate([trio, pad], axis=2)
    return blocks.reshape(_NLIVE * _NT * _NCH * _IBR, _CH)


def kernel(x, d_indices, d_values, weight, filt, bias):
    pad = ((0, 0), (0, _EPTOT - _NNZ))
    rows_p = jnp.pad(d_indices[1:, 0, :], pad)
    cols_p = jnp.pad(d_indices[1:, 1, :], pad)
    vals_p = jnp.pad(d_values[1:], pad)

    idx1 = _pack_idx(rows_p, cols_p, vals_p)
    off = (jnp.arange(_NLIVE, dtype=jnp.int32) * _NP)[:, None]
    idx2 = _pack_idx(rows_p, cols_p + off, vals_p)

    filt4 = filt.reshape(-1, _N)
    filtp = jnp.pad(filt4[1:], ((0, 0), (0, _NP - _N))).reshape(-1, 1)

    h = _matmul(x, weight)
    yp = _sc_stage(h, idx1, (2, _NLIVE, _NP, _D), _dump_per_op)
    ym = _merge_scale(yp.reshape(2, _NLIVE * _NP, _D), filtp)
    op = _sc_stage(ym, idx2, (2, _NP, _D), _dump_final)
    return _final(op, bias.reshape(1, _D))


# speedup vs baseline: 2.1148x; 1.0413x over previous
import functools

import numpy as np

import jax
import jax.numpy as jnp
from jax import lax
from jax.experimental import pallas as pl
from jax.experimental.pallas import tpu as pltpu
from jax.experimental.pallas import tpu_sc as plsc

_N = 10000
_D = 128
_NNZ = 160000
_NLIVE = 3
_NT = 32
_CH = 128
_EPT = 5120
_EPTOT = _EPT * _NT
_NCH = _EPT // _CH
_NP = 10240
_RPS = _NP // 16
_DCH = 128
_IBR = 8


def _mm_body(x_ref, w_ref, o_ref):
    o_ref[:, :] = jnp.dot(x_ref[:, :], w_ref[:, :],
                          preferred_element_type=jnp.float32)


def _matmul(x, w):
    return pl.pallas_call(
        _mm_body,
        grid=(10,),
        in_specs=[pl.BlockSpec((_N // 10, _D), lambda i: (i, 0)),
                  pl.BlockSpec((_D, _D), lambda i: (0, 0))],
        out_specs=pl.BlockSpec((_N // 10, _D), lambda i: (i, 0)),
        out_shape=jax.ShapeDtypeStruct((_N, _D), jnp.float32),
    )(x, w)


def _merge_body(a_ref, f_ref, o_ref):
    o_ref[:, :] = (a_ref[0] + a_ref[1]) * f_ref[:, :]


def _merge_scale(yp, filtp):
    rows = _NLIVE * _NP
    blk = 1024
    return pl.pallas_call(
        _merge_body,
        grid=(rows // blk,),
        in_specs=[pl.BlockSpec((2, blk, _D), lambda i: (0, i, 0)),
                  pl.BlockSpec((blk, 1), lambda i: (i, 0))],
        out_specs=pl.BlockSpec((blk, _D), lambda i: (i, 0)),
        out_shape=jax.ShapeDtypeStruct((rows, _D), jnp.float32),
    )(yp, filtp)


def _final_body(a_ref, b_ref, o_ref):
    o_ref[:, :] = a_ref[0] + a_ref[1] + b_ref[:, :]


def _final(op, bias2d):
    blk = 1000
    return pl.pallas_call(
        _final_body,
        grid=(_N // blk,),
        in_specs=[pl.BlockSpec((2, blk, _D), lambda i: (0, i, 0)),
                  pl.BlockSpec((1, _D), lambda i: (0, 0))],
        out_specs=pl.BlockSpec((blk, _D), lambda i: (i, 0)),
        out_shape=jax.ShapeDtypeStruct((_N, _D), jnp.float32),
    )(op, bias2d)


_MESH = dict(core_axis_name="c", subcore_axis_name="s")
_SC_PARAMS = dict(
    compiler_params=pltpu.CompilerParams(needs_layout_passes=False))


def _zero_buf(buf):
    def zrow(i, carry):
        for j in range(_D // 16):
            buf[i, pl.ds(j * 16, 16)] = jnp.zeros((16,), jnp.float32)
        return carry
    lax.fori_loop(0, _DCH, zrow, 0)


def _scale_rows(gbuf, ibuf):
    def erow(t, carry):
        e = 2 * t
        for u in range(2):
            e16 = jnp.full((16,), u, jnp.int32) + e
            vi = plsc.load_gather(ibuf.at[2], [e16])
            s16 = plsc.bitcast(vi, jnp.float32)
            for g in range(_D // 16):
                gbuf[e + u, pl.ds(g * 16, 16)] = (
                    gbuf[e + u, pl.ds(g * 16, 16)] * s16)
        return carry
    lax.fori_loop(0, _CH // 2, erow, 0)


def _sc_bufs():
    return [
        *[pltpu.VMEM((_IBR, _CH), jnp.int32) for _ in range(4)],
        pltpu.VMEM((_CH, _D), jnp.float32),
        pltpu.VMEM((_CH, _D), jnp.float32),
        *[pltpu.SemaphoreType.DMA for _ in range(6)],
    ]


def _edge_pipeline(src_hbm, idx_hbm, acc, blk0, ibufs, gbufs, isems, gsems):
    ib = ibufs
    gb = gbufs

    def idx_src(c):
        return idx_hbm.at[pl.ds((blk0 + c) * _IBR, _IBR)]

    def load_idx(c, j):
        pltpu.async_copy(idx_src(c), ib[j], isems[j])

    def drain_idx(j):
        pltpu.make_async_copy(idx_hbm.at[pl.ds(0, _IBR)], ib[j],
                              isems[j]).wait()

    def issue_gather(j, p):
        pltpu.async_copy(src_hbm.at[ib[j].at[1]], gb[p], gsems[p])

    def drain_gather(p):
        pltpu.make_async_copy(src_hbm.at[pl.ds(0, _CH)], gb[p],
                              gsems[p]).wait()

    def process(j, p):
        _scale_rows(gb[p], ib[j])
        pltpu.sync_copy(gb[p], acc.at[ib[j].at[0]], add=True)

    pltpu.sync_copy(idx_src(0), ib[0])
    issue_gather(0, 0)
    pltpu.sync_copy(idx_src(1), ib[1])
    issue_gather(1, 1)
    load_idx(2, 2)
    load_idx(3, 3)

    def quad(q, carry):
        c0 = 4 * q
        for j in range(4):
            drain_gather(j % 2)
            process(j, j % 2)
            load_idx(c0 + j + 4, j)
            drain_idx((j + 2) % 4)
            issue_gather((j + 2) % 4, j % 2)
        return carry

    lax.fori_loop(0, _NCH // 4 - 1, quad, 0)

    for j in range(4):
        drain_gather(j % 2)
        process(j, j % 2)
        if j < 2:
            drain_idx(j + 2)
            issue_gather(j + 2, j % 2)


def _sc_stage(src, idx, out_shape, dump):
    mesh = plsc.VectorSubcoreMesh(**_MESH)

    @functools.partial(
        pl.kernel,
        out_type=jax.ShapeDtypeStruct(out_shape, jnp.float32),
        mesh=mesh,
        scratch_types=[
            pltpu.VMEM_SHARED((_NP, _D), jnp.float32),
            *_sc_bufs(),
        ],
        **_SC_PARAMS,
    )
    def k(src_hbm, idx_hbm, out_hbm, acc,
          i0, i1, i2, i3, g0, g1, is0, is1, is2, is3, gs0, gs1):
        cid = lax.axis_index("c")
        sid = lax.axis_index("s")
        tid = cid * 16 + sid

        _zero_buf(g0)
        for i in range(_RPS // _DCH):
            pltpu.sync_copy(g0, acc.at[pl.ds(sid * _RPS + i * _DCH, _DCH)])
        plsc.subcore_barrier()

        for mm in range(_NLIVE):
            blk0 = (mm * _NT + tid) * _NCH
            _edge_pipeline(src_hbm, idx_hbm, acc, blk0,
                           (i0, i1, i2, i3), (g0, g1),
                           (is0, is1, is2, is3), (gs0, gs1))
            plsc.subcore_barrier()
            dump(mm, cid, sid, acc, g0, out_hbm)
            plsc.subcore_barrier()

    return k(src, idx)


def _dump_per_op(mm, cid, sid, acc, zbuf, out_hbm):
    _zero_buf(zbuf)
    for i in range(_RPS // _DCH):
        start = sid * _RPS + i * _DCH
        pltpu.sync_copy(acc.at[pl.ds(start, _DCH)],
                        out_hbm.at[cid, mm, pl.ds(start, _DCH)])
        pltpu.sync_copy(zbuf, acc.at[pl.ds(start, _DCH)])


def _dump_final(mm, cid, sid, acc, zbuf, out_hbm):
    if mm == _NLIVE - 1:
        for i in range(_RPS // _DCH):
            start = sid * _RPS + i * _DCH
            pltpu.sync_copy(acc.at[pl.ds(start, _DCH)],
                            out_hbm.at[cid, pl.ds(start, _DCH)])


def _pack_idx(rows, cols, vals):
    vbits = lax.bitcast_convert_type(vals, jnp.int32)
    trio = jnp.stack([rows, cols, vbits], axis=1)
    trio = trio.reshape(_NLIVE, 3, _NT * _NCH, _CH)
    trio = trio.transpose(0, 2, 1, 3)
    pad = jnp.zeros((_NLIVE, _NT * _NCH, _IBR - 3, _CH), jnp.int32)
    blocks = jnp.concatenate([trio, pad], axis=2)
    return blocks.reshape(_NLIVE * _NT * _NCH * _IBR, _CH)


def kernel(x, d_indices, d_values, weight, filt, bias):
    pad = ((0, 0), (0, _EPTOT - _NNZ))
    rows_p = jnp.pad(d_indices[1:, 0, :], pad)
    cols_p = jnp.pad(d_indices[1:, 1, :], pad)
    vals_p = jnp.pad(d_values[1:], pad)

    idx1 = _pack_idx(rows_p, cols_p, vals_p)
    off = (jnp.arange(_NLIVE, dtype=jnp.int32) * _NP)[:, None]
    idx2 = _pack_idx(rows_p, cols_p + off, vals_p)

    filt4 = filt.reshape(-1, _N)
    filtp = jnp.pad(filt4[1:], ((0, 0), (0, _NP - _N))).reshape(-1, 1)

    h = _matmul(x, weight)
    yp = _sc_stage(h, idx1, (2, _NLIVE, _NP, _D), _dump_per_op)
    ym = _merge_scale(yp.reshape(2, _NLIVE * _NP, _D), filtp)
    op = _sc_stage(ym, idx2, (2, _NP, _D), _dump_final)
    return _final(op, bias.reshape(1, _D))


# speedup vs baseline: 2.1282x; 1.0064x over previous
import functools

import numpy as np

import jax
import jax.numpy as jnp
from jax import lax
from jax.experimental import pallas as pl
from jax.experimental.pallas import tpu as pltpu
from jax.experimental.pallas import tpu_sc as plsc

_N = 10000
_D = 128
_NNZ = 160000
_NLIVE = 3
_NT = 32
_CH = 128
_EPT = 5120
_EPTOT = _EPT * _NT
_NCH = _EPT // _CH
_NP = 10240
_RPS = _NP // 16
_DCH = 128
_IBR = 8


def _mm_body(x_ref, w_ref, o_ref):
    o_ref[:, :] = jnp.dot(x_ref[:, :], w_ref[:, :],
                          preferred_element_type=jnp.float32)


def _matmul(x, w):
    return pl.pallas_call(
        _mm_body,
        grid=(10,),
        in_specs=[pl.BlockSpec((_N // 10, _D), lambda i: (i, 0)),
                  pl.BlockSpec((_D, _D), lambda i: (0, 0))],
        out_specs=pl.BlockSpec((_N // 10, _D), lambda i: (i, 0)),
        out_shape=jax.ShapeDtypeStruct((_N, _D), jnp.float32),
    )(x, w)


def _merge_body(a_ref, f_ref, o_ref):
    o_ref[:, :] = (a_ref[0] + a_ref[1]) * f_ref[:, :]


def _merge_scale(yp, filtp):
    rows = _NLIVE * _NP
    blk = 1024
    return pl.pallas_call(
        _merge_body,
        grid=(rows // blk,),
        in_specs=[pl.BlockSpec((2, blk, _D), lambda i: (0, i, 0)),
                  pl.BlockSpec((blk, 1), lambda i: (i, 0))],
        out_specs=pl.BlockSpec((blk, _D), lambda i: (i, 0)),
        out_shape=jax.ShapeDtypeStruct((rows, _D), jnp.float32),
    )(yp, filtp)


def _final_body(a_ref, b_ref, o_ref):
    o_ref[:, :] = a_ref[0] + a_ref[1] + b_ref[:, :]


def _final(op, bias2d):
    blk = 1000
    return pl.pallas_call(
        _final_body,
        grid=(_N // blk,),
        in_specs=[pl.BlockSpec((2, blk, _D), lambda i: (0, i, 0)),
                  pl.BlockSpec((1, _D), lambda i: (0, 0))],
        out_specs=pl.BlockSpec((blk, _D), lambda i: (i, 0)),
        out_shape=jax.ShapeDtypeStruct((_N, _D), jnp.float32),
    )(op, bias2d)


_MESH = dict(core_axis_name="c", subcore_axis_name="s")
_SC_PARAMS = dict(
    compiler_params=pltpu.CompilerParams(needs_layout_passes=False))


def _zero_buf(buf):
    def zrow(i, carry):
        for j in range(_D // 16):
            buf[i, pl.ds(j * 16, 16)] = jnp.zeros((16,), jnp.float32)
        return carry
    lax.fori_loop(0, _DCH, zrow, 0)


def _scale_rows(gbuf, ibuf):
    def erow(t, carry):
        e = 4 * t
        for u in range(4):
            e16 = jnp.full((16,), u, jnp.int32) + e
            vi = plsc.load_gather(ibuf.at[2], [e16])
            s16 = plsc.bitcast(vi, jnp.float32)
            for g in range(_D // 16):
                gbuf[e + u, pl.ds(g * 16, 16)] = (
                    gbuf[e + u, pl.ds(g * 16, 16)] * s16)
        return carry
    lax.fori_loop(0, _CH // 4, erow, 0)


def _sc_bufs():
    return [
        *[pltpu.VMEM((_IBR, _CH), jnp.int32) for _ in range(4)],
        pltpu.VMEM((_CH, _D), jnp.float32),
        pltpu.VMEM((_CH, _D), jnp.float32),
        *[pltpu.SemaphoreType.DMA for _ in range(6)],
    ]


def _edge_pipeline(src_hbm, idx_hbm, acc, blk0, ibufs, gbufs, isems, gsems):
    ib = ibufs
    gb = gbufs

    def idx_src(c):
        return idx_hbm.at[pl.ds((blk0 + c) * _IBR, _IBR)]

    def load_idx(c, j):
        pltpu.async_copy(idx_src(c), ib[j], isems[j])

    def drain_idx(j):
        pltpu.make_async_copy(idx_hbm.at[pl.ds(0, _IBR)], ib[j],
                              isems[j]).wait()

    def issue_gather(j, p):
        pltpu.async_copy(src_hbm.at[ib[j].at[1]], gb[p], gsems[p])

    def drain_gather(p):
        pltpu.make_async_copy(src_hbm.at[pl.ds(0, _CH)], gb[p],
                              gsems[p]).wait()

    def process(j, p):
        _scale_rows(gb[p], ib[j])
        pltpu.sync_copy(gb[p], acc.at[ib[j].at[0]], add=True)

    pltpu.sync_copy(idx_src(0), ib[0])
    issue_gather(0, 0)
    pltpu.sync_copy(idx_src(1), ib[1])
    issue_gather(1, 1)
    load_idx(2, 2)
    load_idx(3, 3)

    def quad(q, carry):
        c0 = 4 * q
        for j in range(4):
            drain_gather(j % 2)
            process(j, j % 2)
            load_idx(c0 + j + 4, j)
            drain_idx((j + 2) % 4)
            issue_gather((j + 2) % 4, j % 2)
        return carry

    lax.fori_loop(0, _NCH // 4 - 1, quad, 0)

    for j in range(4):
        drain_gather(j % 2)
        process(j, j % 2)
        if j < 2:
            drain_idx(j + 2)
            issue_gather(j + 2, j % 2)


def _sc_stage(src, idx, out_shape, dump):
    mesh = plsc.VectorSubcoreMesh(**_MESH)

    @functools.partial(
        pl.kernel,
        out_type=jax.ShapeDtypeStruct(out_shape, jnp.float32),
        mesh=mesh,
        scratch_types=[
            pltpu.VMEM_SHARED((_NP, _D), jnp.float32),
            *_sc_bufs(),
        ],
        **_SC_PARAMS,
    )
    def k(src_hbm, idx_hbm, out_hbm, acc,
          i0, i1, i2, i3, g0, g1, is0, is1, is2, is3, gs0, gs1):
        cid = lax.axis_index("c")
        sid = lax.axis_index("s")
        tid = cid * 16 + sid

        _zero_buf(g0)
        for i in range(_RPS // _DCH):
            pltpu.sync_copy(g0, acc.at[pl.ds(sid * _RPS + i * _DCH, _DCH)])
        plsc.subcore_barrier()

        for mm in range(_NLIVE):
            blk0 = (mm * _NT + tid) * _NCH
            _edge_pipeline(src_hbm, idx_hbm, acc, blk0,
                           (i0, i1, i2, i3), (g0, g1),
                           (is0, is1, is2, is3), (gs0, gs1))
            plsc.subcore_barrier()
            dump(mm, cid, sid, acc, g0, out_hbm)
            plsc.subcore_barrier()

    return k(src, idx)


def _dump_per_op(mm, cid, sid, acc, zbuf, out_hbm):
    _zero_buf(zbuf)
    for i in range(_RPS // _DCH):
        start = sid * _RPS + i * _DCH
        pltpu.sync_copy(acc.at[pl.ds(start, _DCH)],
                        out_hbm.at[cid, mm, pl.ds(start, _DCH)])
        pltpu.sync_copy(zbuf, acc.at[pl.ds(start, _DCH)])


def _dump_final(mm, cid, sid, acc, zbuf, out_hbm):
    if mm == _NLIVE - 1:
        for i in range(_RPS // _DCH):
            start = sid * _RPS + i * _DCH
            pltpu.sync_copy(acc.at[pl.ds(start, _DCH)],
                            out_hbm.at[cid, pl.ds(start, _DCH)])


def _pack_idx(rows, cols, vals):
    vbits = lax.bitcast_convert_type(vals, jnp.int32)
    trio = jnp.stack([rows, cols, vbits], axis=1)
    trio = trio.reshape(_NLIVE, 3, _NT * _NCH, _CH)
    trio = trio.transpose(0, 2, 1, 3)
    pad = jnp.zeros((_NLIVE, _NT * _NCH, _IBR - 3, _CH), jnp.int32)
    blocks = jnp.concatenate([trio, pad], axis=2)
    return blocks.reshape(_NLIVE * _NT * _NCH * _IBR, _CH)


def kernel(x, d_indices, d_values, weight, filt, bias):
    pad = ((0, 0), (0, _EPTOT - _NNZ))
    rows_p = jnp.pad(d_indices[1:, 0, :], pad)
    cols_p = jnp.pad(d_indices[1:, 1, :], pad)
    vals_p = jnp.pad(d_values[1:], pad)

    idx1 = _pack_idx(rows_p, cols_p, vals_p)
    off = (jnp.arange(_NLIVE, dtype=jnp.int32) * _NP)[:, None]
    idx2 = _pack_idx(rows_p, cols_p + off, vals_p)

    filt4 = filt.reshape(-1, _N)
    filtp = jnp.pad(filt4[1:], ((0, 0), (0, _NP - _N))).reshape(-1, 1)

    h = _matmul(x, weight)
    yp = _sc_stage(h, idx1, (2, _NLIVE, _NP, _D), _dump_per_op)
    ym = _merge_scale(yp.reshape(2, _NLIVE * _NP, _D), filtp)
    op = _sc_stage(ym, idx2, (2, _NP, _D), _dump_final)
    return _final(op, bias.reshape(1, _D))
